# baseline (speedup 1.0000x reference)
import jax
import jax.numpy as jnp
from jax.experimental import pallas as pl
from jax.experimental.pallas import tpu as pltpu

N = 10000
IN_F = 128
OUT_F = 128
BM = 200
HALF = N // 2


def _body(x_ref, adjt_ref, adjb_ref, w_ref, outt_ref, outb_ref, h_ref):
    @pl.when(pl.program_id(0) == 0)
    def _():
        h = jnp.dot(x_ref[...], w_ref[...],
                    preferred_element_type=jnp.float32)
        h_ref[...] = h.astype(jnp.bfloat16)

    outt_ref[...] = jnp.dot(adjt_ref[...].astype(jnp.bfloat16), h_ref[...],
                            preferred_element_type=jnp.float32)
    outb_ref[...] = jnp.dot(adjb_ref[...].astype(jnp.bfloat16), h_ref[...],
                            preferred_element_type=jnp.float32)


def kernel(x, adj, W, a):
    del a
    nblk = HALF // BM
    out_t, out_b = pl.pallas_call(
        _body,
        grid=(nblk,),
        in_specs=[
            pl.BlockSpec((N, IN_F), lambda i: (0, 0)),
            pl.BlockSpec((BM, N), lambda i: (i, 0)),
            pl.BlockSpec((BM, N), lambda i: (i + nblk, 0)),
            pl.BlockSpec((IN_F, OUT_F), lambda i: (0, 0)),
        ],
        out_specs=[
            pl.BlockSpec((BM, OUT_F), lambda i: (i, 0)),
            pl.BlockSpec((BM, OUT_F), lambda i: (i, 0)),
        ],
        out_shape=[
            jax.ShapeDtypeStruct((HALF, OUT_F), jnp.float32),
            jax.ShapeDtypeStruct((HALF, OUT_F), jnp.float32),
        ],
        scratch_shapes=[pltpu.VMEM((N, OUT_F), jnp.bfloat16)],
        compiler_params=pltpu.CompilerParams(
            dimension_semantics=("arbitrary",),
        ),
    )(x, adj, adj, W)
    return jnp.concatenate([out_t, out_b], axis=0)
